# baseline (speedup 1.0000x reference)
import functools

import jax
import jax.numpy as jnp
from jax import lax
from jax.experimental import pallas as pl
from jax.experimental.pallas import tpu as pltpu
from jax.experimental.pallas import tpu_sc as plsc

H = 768
F = 768
E = 64
T = 256
L = 16
NWORK = T // L


def _logits_body(x_ref, gw_ref, lt_ref):
    lt_ref[...] = lax.dot_general(
        gw_ref[...], x_ref[...],
        dimension_numbers=(((0,), (1,)), ((), ())),
        preferred_element_type=jnp.float32,
    )


def _routing_sc_body(lt_hbm, comb_hbm, lt_v, cvt):
    wid = lax.axis_index("s")

    @pl.when(wid < NWORK)
    def _():
        pltpu.sync_copy(lt_hbm, lt_v)
        base = wid * L
        neg = jnp.full((L,), -3.0e38, jnp.float32)
        m1, m2 = neg, neg
        i1 = jnp.zeros((L,), jnp.int32)
        i2 = jnp.zeros((L,), jnp.int32)
        for e in range(E):
            v = lt_v[e, pl.ds(base, L)]
            gt1 = v > m1
            gt2 = v > m2
            i2 = jnp.where(gt1, i1, jnp.where(gt2, e, i2))
            m2 = jnp.where(gt1, m1, jnp.where(gt2, v, m2))
            i1 = jnp.where(gt1, e, i1)
            m1 = jnp.where(gt1, v, m1)
        w1 = 1.0 / (1.0 + jnp.exp(m2 - m1))
        ids = lax.iota(jnp.int32, L)
        gdn = lax.GatherDimensionNumbers(
            offset_dims=(), collapsed_slice_dims=(0,), start_index_map=(0,))

        def _bcast(v, t):
            idx = jnp.full((L, 1), t, jnp.int32)
            return lax.gather(v, idx, gdn, slice_sizes=(1,),
                              mode=lax.GatherScatterMode.PROMISE_IN_BOUNDS)

        for t in range(L):
            i1b = _bcast(i1, t)
            i2b = _bcast(i2, t)
            w1b = _bcast(w1, t)
            for c in range(E // L):
                ce = ids + (c * L)
                cvt[t, pl.ds(c * L, L)] = jnp.where(
                    ce == i1b, w1b, jnp.where(ce == i2b, 1.0 - w1b, 0.0))
        pltpu.sync_copy(cvt, comb_hbm.at[pl.ds(wid * L, L), :])


_routing_sc = functools.partial(
    pl.kernel,
    out_type=jax.ShapeDtypeStruct((T, E), jnp.float32),
    mesh=plsc.VectorSubcoreMesh(core_axis_name="c", subcore_axis_name="s",
                                num_cores=1),
    scratch_types=[
        pltpu.VMEM((E, T), jnp.float32),
        pltpu.VMEM((L, E), jnp.float32),
    ],
)(_routing_sc_body)


NE = 2


def _moe_body(xb_ref, comb_ref, wg_ref, wu_ref, wd_ref, out_ref):
    blk = pl.program_id(0)
    xb = xb_ref[...]
    eids = blk * NE + lax.broadcasted_iota(jnp.int32, (1, NE), 1)
    onehot = (lax.broadcasted_iota(jnp.int32, (E, NE), 0) == eids
              ).astype(jnp.float32)
    ce = lax.dot(comb_ref[...], onehot,
                 preferred_element_type=jnp.float32)
    acc = jnp.zeros_like(out_ref)
    for j in range(NE):
        wg = wg_ref[j].astype(jnp.bfloat16)
        wu = wu_ref[j].astype(jnp.bfloat16)
        g = lax.dot(xb, wg, preferred_element_type=jnp.float32)
        u = lax.dot(xb, wu, preferred_element_type=jnp.float32)
        h = (g * lax.logistic(g)) * u
        wd = wd_ref[j].astype(jnp.bfloat16)
        y = lax.dot(h.astype(jnp.bfloat16), wd,
                    preferred_element_type=jnp.float32)
        acc = acc + ce[:, j:j + 1] * y

    @pl.when(blk == 0)
    def _():
        out_ref[...] = acc

    @pl.when(blk != 0)
    def _():
        out_ref[...] += acc


def kernel(hidden_states, gate_w, Wg, Wu, Wd):
    x = hidden_states.reshape(-1, H)

    logitsT = pl.pallas_call(
        _logits_body,
        out_shape=jax.ShapeDtypeStruct((E, T), jnp.float32),
    )(x, gate_w)

    comb = _routing_sc(logitsT)

    xb = x.astype(jnp.bfloat16)
    out = pl.pallas_call(
        _moe_body,
        grid=(E // NE,),
        in_specs=[
            pl.BlockSpec((T, H), lambda b: (0, 0)),
            pl.BlockSpec((T, E), lambda b: (0, 0)),
            pl.BlockSpec((NE, H, F), lambda b: (b, 0, 0)),
            pl.BlockSpec((NE, H, F), lambda b: (b, 0, 0)),
            pl.BlockSpec((NE, F, H), lambda b: (b, 0, 0)),
        ],
        out_specs=pl.BlockSpec((T, H), lambda b: (0, 0)),
        out_shape=jax.ShapeDtypeStruct((T, H), jnp.float32),
        compiler_params=pltpu.CompilerParams(
            dimension_semantics=("arbitrary",),
        ),
    )(xb, comb, Wg, Wu, Wd)
    return out


# speedup vs baseline: 1.0062x; 1.0062x over previous
import functools

import jax
import jax.numpy as jnp
from jax import lax
from jax.experimental import pallas as pl
from jax.experimental.pallas import tpu as pltpu
from jax.experimental.pallas import tpu_sc as plsc

H = 768
F = 768
E = 64
T = 256
L = 16
NWORK = T // L


def _logits_body(x_ref, gw_ref, lt_ref):
    lt_ref[...] = lax.dot_general(
        gw_ref[...], x_ref[...],
        dimension_numbers=(((0,), (1,)), ((), ())),
        preferred_element_type=jnp.float32,
    )


def _routing_sc_body(lt_hbm, comb_hbm, lt_v, cvt):
    cid = lax.axis_index("c")
    sid = lax.axis_index("s")
    wid = sid * 2 + cid

    @pl.when(wid < NWORK)
    def _():
        pltpu.sync_copy(lt_hbm, lt_v)
        base = wid * L
        neg = jnp.full((L,), -3.0e38, jnp.float32)
        m1, m2 = neg, neg
        i1 = jnp.zeros((L,), jnp.int32)
        i2 = jnp.zeros((L,), jnp.int32)
        for e in range(E):
            v = lt_v[e, pl.ds(base, L)]
            gt1 = v > m1
            gt2 = v > m2
            i2 = jnp.where(gt1, i1, jnp.where(gt2, e, i2))
            m2 = jnp.where(gt1, m1, jnp.where(gt2, v, m2))
            i1 = jnp.where(gt1, e, i1)
            m1 = jnp.where(gt1, v, m1)
        w1 = 1.0 / (1.0 + jnp.exp(m2 - m1))
        ids = lax.iota(jnp.int32, L)
        gdn = lax.GatherDimensionNumbers(
            offset_dims=(), collapsed_slice_dims=(0,), start_index_map=(0,))

        def _bcast(v, t):
            idx = jnp.full((L, 1), t, jnp.int32)
            return lax.gather(v, idx, gdn, slice_sizes=(1,),
                              mode=lax.GatherScatterMode.PROMISE_IN_BOUNDS)

        for t in range(L):
            i1b = _bcast(i1, t)
            i2b = _bcast(i2, t)
            w1b = _bcast(w1, t)
            for c in range(E // L):
                ce = ids + (c * L)
                cvt[t, pl.ds(c * L, L)] = jnp.where(
                    ce == i1b, w1b, jnp.where(ce == i2b, 1.0 - w1b, 0.0))
        pltpu.sync_copy(cvt, comb_hbm.at[pl.ds(wid * L, L), :])


_routing_sc = functools.partial(
    pl.kernel,
    out_type=jax.ShapeDtypeStruct((T, E), jnp.float32),
    mesh=plsc.VectorSubcoreMesh(core_axis_name="c", subcore_axis_name="s"),
    scratch_types=[
        pltpu.VMEM((E, T), jnp.float32),
        pltpu.VMEM((L, E), jnp.float32),
    ],
)(_routing_sc_body)


NE = 2


def _moe_body(xb_ref, comb_ref, wg_ref, wu_ref, wd_ref, out_ref):
    blk = pl.program_id(0)
    xb = xb_ref[...]
    eids = blk * NE + lax.broadcasted_iota(jnp.int32, (1, NE), 1)
    onehot = (lax.broadcasted_iota(jnp.int32, (E, NE), 0) == eids
              ).astype(jnp.float32)
    ce = lax.dot(comb_ref[...], onehot,
                 preferred_element_type=jnp.float32)
    acc = jnp.zeros_like(out_ref)
    for j in range(NE):
        wg = wg_ref[j].astype(jnp.bfloat16)
        wu = wu_ref[j].astype(jnp.bfloat16)
        g = lax.dot(xb, wg, preferred_element_type=jnp.float32)
        u = lax.dot(xb, wu, preferred_element_type=jnp.float32)
        h = (g * lax.logistic(g)) * u
        wd = wd_ref[j].astype(jnp.bfloat16)
        y = lax.dot(h.astype(jnp.bfloat16), wd,
                    preferred_element_type=jnp.float32)
        acc = acc + ce[:, j:j + 1] * y

    @pl.when(blk == 0)
    def _():
        out_ref[...] = acc

    @pl.when(blk != 0)
    def _():
        out_ref[...] += acc


def kernel(hidden_states, gate_w, Wg, Wu, Wd):
    x = hidden_states.reshape(-1, H)

    logitsT = pl.pallas_call(
        _logits_body,
        out_shape=jax.ShapeDtypeStruct((E, T), jnp.float32),
    )(x, gate_w)

    comb = _routing_sc(logitsT)

    xb = x.astype(jnp.bfloat16)
    out = pl.pallas_call(
        _moe_body,
        grid=(E // NE,),
        in_specs=[
            pl.BlockSpec((T, H), lambda b: (0, 0)),
            pl.BlockSpec((T, E), lambda b: (0, 0)),
            pl.BlockSpec((NE, H, F), lambda b: (b, 0, 0)),
            pl.BlockSpec((NE, H, F), lambda b: (b, 0, 0)),
            pl.BlockSpec((NE, F, H), lambda b: (b, 0, 0)),
        ],
        out_specs=pl.BlockSpec((T, H), lambda b: (0, 0)),
        out_shape=jax.ShapeDtypeStruct((T, H), jnp.float32),
        compiler_params=pltpu.CompilerParams(
            dimension_semantics=("arbitrary",),
        ),
    )(xb, comb, Wg, Wu, Wd)
    return out
